# Initial kernel scaffold; baseline (speedup 1.0000x reference)
import jax
import jax.numpy as jnp
from jax.experimental import pallas as pl


def kernel(x, edge_index, W1, b1, W2, b2, W3, b3):
    raise NotImplementedError("write your pallas kernel here")


# speedup vs baseline: 26.5546x; 26.5546x over previous
import functools

import jax
import jax.numpy as jnp
from jax import lax
from jax.experimental import pallas as pl
from jax.experimental.pallas import tpu as pltpu
from jax.experimental.pallas import tpu_sc as plsc

N = 100000
E = 1600000
HID = 32
OUT = 16
HALF = 16
ROWS = E // 128
NT = 16
NC = 2
CHUNK_E = 1024
RPT = 6256
RPT_LAST = N - 15 * RPT

_mesh = plsc.VectorSubcoreMesh(core_axis_name="c", subcore_axis_name="s")
_sc_params = pltpu.CompilerParams(use_tc_tiling_on_sc=False)


def _zero_acc_2d(rows, acc, t):
    base = t * RPT
    for k in range(6):
        pltpu.sync_copy(rows, acc.at[pl.ds(base + k * CHUNK_E, CHUNK_E)])

    @pl.when(t < 15)
    def _():
        pltpu.sync_copy(rows.at[pl.ds(0, RPT - 6 * CHUNK_E)],
                        acc.at[pl.ds(base + 6 * CHUNK_E, RPT - 6 * CHUNK_E)])

    @pl.when(t == 15)
    def _():
        pltpu.sync_copy(rows.at[pl.ds(0, RPT_LAST - 6 * CHUNK_E)],
                        acc.at[pl.ds(base + 6 * CHUNK_E, RPT_LAST - 6 * CHUNK_E)])


def _writeout_2d(acc, dst, t):
    base = t * RPT

    @pl.when(t < 15)
    def _():
        pltpu.sync_copy(acc.at[pl.ds(base, RPT)], dst.at[pl.ds(base, RPT)])

    @pl.when(t == 15)
    def _():
        pltpu.sync_copy(acc.at[pl.ds(base, RPT_LAST)],
                        dst.at[pl.ds(base, RPT_LAST)])


def _make_agg(split_edges: bool):
    total_e = E // NC if split_edges else E
    nchunks = total_e // CHUNK_E
    tail_e = total_e - nchunks * CHUNK_E
    base_ch = nchunks // NT
    extra_ch = nchunks % NT

    @functools.partial(
        pl.kernel,
        mesh=_mesh,
        out_type=jax.ShapeDtypeStruct((NC, N, HALF), jnp.float32),
        scratch_types=[
            pltpu.VMEM((CHUNK_E,), jnp.int32),
            pltpu.VMEM((CHUNK_E,), jnp.int32),
            pltpu.VMEM((tail_e,), jnp.int32),
            pltpu.VMEM((tail_e,), jnp.int32),
            pltpu.VMEM((CHUNK_E, HALF), jnp.float32),
            pltpu.VMEM_SHARED((N, HALF), jnp.float32),
            pltpu.SemaphoreType.DMA,
        ],
        compiler_params=_sc_params,
    )
    def agg(table, ei, out, sidx, didx, sidx_t, didx_t, rows, acc, gsem):
        c = lax.axis_index("c")
        t = lax.axis_index("s")
        tbl = table if split_edges else table.at[c]
        e0 = c * total_e if split_edges else 0

        def zb(i, carry):
            rows[i, :] = jnp.zeros((HALF,), jnp.float32)
            return carry
        lax.fori_loop(0, CHUNK_E, zb, None)
        _zero_acc_2d(rows, acc, t)
        plsc.subcore_barrier()

        nmine = base_ch + jnp.where(t < extra_ch, 1, 0)

        def chunk(i, carry):
            eoff = e0 + (t + i * NT) * CHUNK_E
            pltpu.sync_copy(ei.at[0].at[pl.ds(eoff, CHUNK_E)], sidx)
            pltpu.sync_copy(ei.at[1].at[pl.ds(eoff, CHUNK_E)], didx)
            pltpu.async_copy(tbl.at[sidx], rows, gsem).wait()
            pltpu.sync_copy(rows, acc.at[didx], add=True)
            return carry
        lax.fori_loop(0, nmine, chunk, None)

        @pl.when(t == NT - 1)
        def _():
            eoff = e0 + nchunks * CHUNK_E
            pltpu.sync_copy(ei.at[0].at[pl.ds(eoff, tail_e)], sidx_t)
            pltpu.sync_copy(ei.at[1].at[pl.ds(eoff, tail_e)], didx_t)
            pltpu.async_copy(tbl.at[sidx_t],
                             rows.at[pl.ds(0, tail_e)], gsem).wait()
            pltpu.sync_copy(rows.at[pl.ds(0, tail_e)],
                            acc.at[didx_t], add=True)

        plsc.subcore_barrier()
        _writeout_2d(acc, out.at[c], t)

    return agg


_agg_feat_split = _make_agg(split_edges=False)
_agg_edge_split = _make_agg(split_edges=True)


_DEG_E = E // NC
_DEG_NCH = _DEG_E // CHUNK_E
_DEG_TAIL = _DEG_E - _DEG_NCH * CHUNK_E


@functools.partial(
    pl.kernel,
    mesh=_mesh,
    out_type=jax.ShapeDtypeStruct((NC, N, HALF), jnp.float32),
    scratch_types=[
        pltpu.VMEM((CHUNK_E,), jnp.int32),
        pltpu.VMEM((_DEG_TAIL,), jnp.int32),
        pltpu.VMEM((CHUNK_E, HALF), jnp.float32),
        pltpu.VMEM_SHARED((N, HALF), jnp.float32),
    ],
    compiler_params=_sc_params,
)
def _deg_kernel(ei, out, didx, didx_t, ones, acc):
    c = lax.axis_index("c")
    t = lax.axis_index("s")
    e0 = c * _DEG_E

    def fill(i, carry):
        ones[i, :] = jnp.zeros((HALF,), jnp.float32)
        return carry
    lax.fori_loop(0, CHUNK_E, fill, None)
    _zero_acc_2d(ones, acc, t)
    plsc.subcore_barrier()

    def refill(i, carry):
        ones[i, :] = jnp.ones((HALF,), jnp.float32)
        return carry
    lax.fori_loop(0, CHUNK_E, refill, None)

    nmine = _DEG_NCH // NT + jnp.where(t < _DEG_NCH % NT, 1, 0)

    def chunk(i, carry):
        eoff = e0 + (t + i * NT) * CHUNK_E
        pltpu.sync_copy(ei.at[1].at[pl.ds(eoff, CHUNK_E)], didx)
        pltpu.sync_copy(ones, acc.at[didx], add=True)
        return carry
    lax.fori_loop(0, nmine, chunk, None)

    @pl.when(t == NT - 1)
    def _():
        eoff = e0 + _DEG_NCH * CHUNK_E
        pltpu.sync_copy(ei.at[1].at[pl.ds(eoff, _DEG_TAIL)], didx_t)
        pltpu.sync_copy(ones.at[pl.ds(0, _DEG_TAIL)],
                        acc.at[didx_t], add=True)

    plsc.subcore_barrier()
    _writeout_2d(acc, out.at[c], t)


_NB = 2000
_GRID = N // _NB


def _tc1_body(x_ref, degp_ref, w1_ref, y_ref, dinv_ref):
    deg = degp_ref[0] + degp_ref[1] + 1.0
    dinv = lax.rsqrt(deg)
    dinv_ref[...] = dinv
    h = jnp.dot(x_ref[...], w1_ref[...], preferred_element_type=jnp.float32)
    y = dinv * h
    y_ref[0] = y[:, :HALF]
    y_ref[1] = y[:, HALF:]


def _tc_mid_body(agg_ref, yp_ref, dinv_ref, b_ref, w_ref, y_ref):
    dinv = dinv_ref[...]
    z = jnp.concatenate([agg_ref[0] + yp_ref[0], agg_ref[1] + yp_ref[1]],
                        axis=1)
    a = jnp.maximum(dinv * z + b_ref[...], 0.0)
    y = dinv * jnp.dot(a, w_ref[...], preferred_element_type=jnp.float32)
    y_ref[0] = y[:, :HALF]
    y_ref[1] = y[:, HALF:]


def _tc_last_body(agg_ref, yp_ref, dinv_ref, b_ref, w_ref, y_ref):
    dinv = dinv_ref[...]
    z = jnp.concatenate([agg_ref[0] + yp_ref[0], agg_ref[1] + yp_ref[1]],
                        axis=1)
    a = jnp.maximum(dinv * z + b_ref[...], 0.0)
    y_ref[...] = dinv * jnp.dot(a, w_ref[...],
                                preferred_element_type=jnp.float32)


def _tc_out_body(agg_ref, y3_ref, dinv_ref, b_ref, out_ref):
    out_ref[...] = (dinv_ref[...] * (agg_ref[0] + agg_ref[1] + y3_ref[...])
                    + b_ref[...])


def _bs2(shape):
    return pl.BlockSpec((NC, _NB) + shape[2:], lambda i: (0, i) + (0,) * (len(shape) - 2))


def _bsn(shape):
    return pl.BlockSpec((_NB,) + shape[1:], lambda i: (i,) + (0,) * (len(shape) - 1))


def _bsfull(shape):
    return pl.BlockSpec(shape, lambda i: (0,) * len(shape))


_tc1 = pl.pallas_call(
    _tc1_body,
    grid=(_GRID,),
    in_specs=[_bsn((N, 3)), _bs2((NC, N, 1)), _bsfull((3, HID))],
    out_specs=[_bs2((NC, N, HALF)), _bsn((N, 1))],
    out_shape=[jax.ShapeDtypeStruct((NC, N, HALF), jnp.float32),
               jax.ShapeDtypeStruct((N, 1), jnp.float32)],
)

_tc_mid = pl.pallas_call(
    _tc_mid_body,
    grid=(_GRID,),
    in_specs=[_bs2((NC, N, HALF)), _bs2((NC, N, HALF)), _bsn((N, 1)),
              _bsfull((1, HID)), _bsfull((HID, HID))],
    out_specs=_bs2((NC, N, HALF)),
    out_shape=jax.ShapeDtypeStruct((NC, N, HALF), jnp.float32),
)

_tc_last = pl.pallas_call(
    _tc_last_body,
    grid=(_GRID,),
    in_specs=[_bs2((NC, N, HALF)), _bs2((NC, N, HALF)), _bsn((N, 1)),
              _bsfull((1, HID)), _bsfull((HID, OUT))],
    out_specs=_bsn((N, OUT)),
    out_shape=jax.ShapeDtypeStruct((N, OUT), jnp.float32),
)

_tc_out = pl.pallas_call(
    _tc_out_body,
    grid=(_GRID,),
    in_specs=[_bs2((NC, N, HALF)), _bsn((N, OUT)), _bsn((N, 1)),
              _bsfull((1, OUT))],
    out_specs=_bsn((N, OUT)),
    out_shape=jax.ShapeDtypeStruct((N, OUT), jnp.float32),
)


def kernel(x, edge_index, W1, b1, W2, b2, W3, b3):
    deg16 = _deg_kernel(edge_index)
    degp = deg16[:, :, :1]
    y1, dinv = _tc1(x, degp, W1)
    agg1 = _agg_feat_split(y1, edge_index)
    y2 = _tc_mid(agg1, y1, dinv, b1.reshape(1, HID), W2)
    agg2 = _agg_feat_split(y2, edge_index)
    y3 = _tc_last(agg2, y2, dinv, b2.reshape(1, HID), W3)
    agg3 = _agg_edge_split(y3, edge_index)
    return _tc_out(agg3, y3, dinv, b3.reshape(1, OUT))


# speedup vs baseline: 28.9033x; 1.0885x over previous
import functools

import jax
import jax.numpy as jnp
from jax import lax
from jax.experimental import pallas as pl
from jax.experimental.pallas import tpu as pltpu
from jax.experimental.pallas import tpu_sc as plsc

N = 100000
E = 1600000
HID = 32
OUT = 16
HALF = 16
ROWS = E // 128
NT = 16
NC = 2
CHUNK_E = 400
RPT = 6256
RPT_LAST = N - 15 * RPT

_mesh = plsc.VectorSubcoreMesh(core_axis_name="c", subcore_axis_name="s")
_sc_params = pltpu.CompilerParams(use_tc_tiling_on_sc=False)


def _zero_acc_2d(rows, acc, t):
    base = t * RPT
    kf = RPT_LAST // CHUNK_E
    for k in range(kf):
        pltpu.sync_copy(rows, acc.at[pl.ds(base + k * CHUNK_E, CHUNK_E)])

    @pl.when(t < 15)
    def _():
        pltpu.sync_copy(rows.at[pl.ds(0, RPT - kf * CHUNK_E)],
                        acc.at[pl.ds(base + kf * CHUNK_E, RPT - kf * CHUNK_E)])

    @pl.when(t == 15)
    def _():
        pltpu.sync_copy(rows.at[pl.ds(0, RPT_LAST - kf * CHUNK_E)],
                        acc.at[pl.ds(base + kf * CHUNK_E,
                                     RPT_LAST - kf * CHUNK_E)])


def _writeout_2d(acc, dst, t):
    base = t * RPT

    @pl.when(t < 15)
    def _():
        pltpu.sync_copy(acc.at[pl.ds(base, RPT)], dst.at[pl.ds(base, RPT)])

    @pl.when(t == 15)
    def _():
        pltpu.sync_copy(acc.at[pl.ds(base, RPT_LAST)],
                        dst.at[pl.ds(base, RPT_LAST)])


def _make_agg(split_edges: bool):
    total_e = E // NC if split_edges else E
    nchunks = total_e // CHUNK_E
    assert nchunks * CHUNK_E == total_e
    base_ch = nchunks // NT
    extra_ch = nchunks % NT

    @functools.partial(
        pl.kernel,
        mesh=_mesh,
        out_type=jax.ShapeDtypeStruct((NC, N, HALF), jnp.float32),
        scratch_types=[
            pltpu.VMEM((2, CHUNK_E), jnp.int32),
            pltpu.VMEM((2, CHUNK_E), jnp.int32),
            pltpu.VMEM((2, CHUNK_E), jnp.int32),
            pltpu.VMEM((CHUNK_E, HALF), jnp.float32),
            pltpu.VMEM((CHUNK_E, HALF), jnp.float32),
            pltpu.VMEM((CHUNK_E, HALF), jnp.float32),
            pltpu.VMEM_SHARED((N, HALF), jnp.float32),
            pltpu.SemaphoreType.DMA,
            pltpu.SemaphoreType.DMA,
            pltpu.SemaphoreType.DMA,
            pltpu.SemaphoreType.DMA,
            pltpu.SemaphoreType.DMA,
            pltpu.SemaphoreType.DMA,
            pltpu.SemaphoreType.DMA,
            pltpu.SemaphoreType.DMA,
            pltpu.SemaphoreType.DMA,
        ],
        compiler_params=_sc_params,
    )
    def agg(table, ei, out, idx0, idx1, idx2, rows0, rows1, rows2, acc,
            xsem0, xsem1, xsem2, gsem0, gsem1, gsem2, ssem0, ssem1, ssem2):
        c = lax.axis_index("c")
        t = lax.axis_index("s")
        tbl = table if split_edges else table.at[c]
        e0 = c * total_e if split_edges else 0

        def zb(i, carry):
            rows0[i, :] = jnp.zeros((HALF,), jnp.float32)
            return carry
        lax.fori_loop(0, CHUNK_E, zb, None)
        _zero_acc_2d(rows0, acc, t)
        plsc.subcore_barrier()

        nmine = base_ch + jnp.where(t < extra_ch, 1, 0)
        SL = ((idx0, rows0, xsem0, gsem0, ssem0),
              (idx1, rows1, xsem1, gsem1, ssem1),
              (idx2, rows2, xsem2, gsem2, ssem2))

        def chunk_off(i):
            return e0 + (t + i * NT) * CHUNK_E

        def load_idx(i, slot):
            pltpu.async_copy(ei.at[:, pl.ds(chunk_off(i), CHUNK_E)],
                             slot[0], slot[2])

        def gather_of(slot):
            return pltpu.make_async_copy(tbl.at[slot[0].at[0]], slot[1],
                                         slot[3])

        def scatter_of(slot):
            return pltpu.make_async_copy(slot[1], acc.at[slot[0].at[1]],
                                         slot[4])

        load_idx(0, SL[0])

        def step(i, cur, prv, nxt):
            @pl.when(i >= 1)
            def _():
                gather_of(prv).wait()
                scatter_of(prv).start(add=True)

            @pl.when(i >= 2)
            def _():
                scatter_of(nxt).wait()

            @pl.when(i + 1 < nmine)
            def _():
                load_idx(i + 1, nxt)

            pltpu.make_async_copy(ei.at[:, pl.ds(0, CHUNK_E)], cur[0],
                                  cur[2]).wait()
            gather_of(cur).start()

        def chunk(i, carry):
            for j in range(3):
                @pl.when(i % 3 == j)
                def _(j=j):
                    step(i, SL[j], SL[(j + 2) % 3], SL[(j + 1) % 3])
            return carry
        lax.fori_loop(0, nmine, chunk, None)

        def fin(cur, prv):
            gather_of(cur).wait()
            scatter_of(cur).start(add=True)

            @pl.when(nmine >= 2)
            def _():
                scatter_of(prv).wait()
            scatter_of(cur).wait()

        for j in range(3):
            @pl.when(nmine % 3 == (j + 1) % 3)
            def _(j=j):
                fin(SL[j], SL[(j + 2) % 3])

        plsc.subcore_barrier()
        _writeout_2d(acc, out.at[c], t)

    return agg


_agg_feat_split = _make_agg(split_edges=False)
_agg_edge_split = _make_agg(split_edges=True)


_DEG_E = E // NC
_DEG_NCH = _DEG_E // CHUNK_E


@functools.partial(
    pl.kernel,
    mesh=_mesh,
    out_type=jax.ShapeDtypeStruct((NC, N, HALF), jnp.float32),
    scratch_types=[
        pltpu.VMEM((CHUNK_E,), jnp.int32),
        pltpu.VMEM((CHUNK_E, HALF), jnp.float32),
        pltpu.VMEM_SHARED((N, HALF), jnp.float32),
    ],
    compiler_params=_sc_params,
)
def _deg_kernel(ei, out, didx0, ones, acc):
    c = lax.axis_index("c")
    t = lax.axis_index("s")
    e0 = c * _DEG_E

    def fill(i, carry):
        ones[i, :] = jnp.zeros((HALF,), jnp.float32)
        return carry
    lax.fori_loop(0, CHUNK_E, fill, None)
    _zero_acc_2d(ones, acc, t)
    plsc.subcore_barrier()

    def refill(i, carry):
        ones[i, :] = jnp.ones((HALF,), jnp.float32)
        return carry
    lax.fori_loop(0, CHUNK_E, refill, None)

    nmine = _DEG_NCH // NT + jnp.where(t < _DEG_NCH % NT, 1, 0)

    def chunk(i, carry):
        eoff = e0 + (t + i * NT) * CHUNK_E
        pltpu.sync_copy(ei.at[1].at[pl.ds(eoff, CHUNK_E)], didx0)
        pltpu.sync_copy(ones, acc.at[didx0], add=True)
        return carry
    lax.fori_loop(0, nmine, chunk, None)

    plsc.subcore_barrier()
    _writeout_2d(acc, out.at[c], t)


_NB = 2000
_GRID = N // _NB


def _tc1_body(x_ref, degp_ref, w1_ref, y_ref, dinv_ref):
    deg = degp_ref[0] + degp_ref[1] + 1.0
    dinv = lax.rsqrt(deg)
    dinv_ref[...] = dinv
    h = jnp.dot(x_ref[...], w1_ref[...], preferred_element_type=jnp.float32)
    y = dinv * h
    y_ref[0] = y[:, :HALF]
    y_ref[1] = y[:, HALF:]


def _tc_mid_body(agg_ref, yp_ref, dinv_ref, b_ref, w_ref, y_ref):
    dinv = dinv_ref[...]
    z = jnp.concatenate([agg_ref[0] + yp_ref[0], agg_ref[1] + yp_ref[1]],
                        axis=1)
    a = jnp.maximum(dinv * z + b_ref[...], 0.0)
    y = dinv * jnp.dot(a, w_ref[...], preferred_element_type=jnp.float32)
    y_ref[0] = y[:, :HALF]
    y_ref[1] = y[:, HALF:]


def _tc_last_body(agg_ref, yp_ref, dinv_ref, b_ref, w_ref, y_ref):
    dinv = dinv_ref[...]
    z = jnp.concatenate([agg_ref[0] + yp_ref[0], agg_ref[1] + yp_ref[1]],
                        axis=1)
    a = jnp.maximum(dinv * z + b_ref[...], 0.0)
    y_ref[...] = dinv * jnp.dot(a, w_ref[...],
                                preferred_element_type=jnp.float32)


def _tc_out_body(agg_ref, y3_ref, dinv_ref, b_ref, out_ref):
    out_ref[...] = (dinv_ref[...] * (agg_ref[0] + agg_ref[1] + y3_ref[...])
                    + b_ref[...])


def _bs2(shape):
    return pl.BlockSpec((NC, _NB) + shape[2:], lambda i: (0, i) + (0,) * (len(shape) - 2))


def _bsn(shape):
    return pl.BlockSpec((_NB,) + shape[1:], lambda i: (i,) + (0,) * (len(shape) - 1))


def _bsfull(shape):
    return pl.BlockSpec(shape, lambda i: (0,) * len(shape))


_tc1 = pl.pallas_call(
    _tc1_body,
    grid=(_GRID,),
    in_specs=[_bsn((N, 3)), _bs2((NC, N, 1)), _bsfull((3, HID))],
    out_specs=[_bs2((NC, N, HALF)), _bsn((N, 1))],
    out_shape=[jax.ShapeDtypeStruct((NC, N, HALF), jnp.float32),
               jax.ShapeDtypeStruct((N, 1), jnp.float32)],
)

_tc_mid = pl.pallas_call(
    _tc_mid_body,
    grid=(_GRID,),
    in_specs=[_bs2((NC, N, HALF)), _bs2((NC, N, HALF)), _bsn((N, 1)),
              _bsfull((1, HID)), _bsfull((HID, HID))],
    out_specs=_bs2((NC, N, HALF)),
    out_shape=jax.ShapeDtypeStruct((NC, N, HALF), jnp.float32),
)

_tc_last = pl.pallas_call(
    _tc_last_body,
    grid=(_GRID,),
    in_specs=[_bs2((NC, N, HALF)), _bs2((NC, N, HALF)), _bsn((N, 1)),
              _bsfull((1, HID)), _bsfull((HID, OUT))],
    out_specs=_bsn((N, OUT)),
    out_shape=jax.ShapeDtypeStruct((N, OUT), jnp.float32),
)

_tc_out = pl.pallas_call(
    _tc_out_body,
    grid=(_GRID,),
    in_specs=[_bs2((NC, N, HALF)), _bsn((N, OUT)), _bsn((N, 1)),
              _bsfull((1, OUT))],
    out_specs=_bsn((N, OUT)),
    out_shape=jax.ShapeDtypeStruct((N, OUT), jnp.float32),
)


def kernel(x, edge_index, W1, b1, W2, b2, W3, b3):
    deg16 = _deg_kernel(edge_index)
    degp = deg16[:, :, :1]
    y1, dinv = _tc1(x, degp, W1)
    agg1 = _agg_feat_split(y1, edge_index)
    y2 = _tc_mid(agg1, y1, dinv, b1.reshape(1, HID), W2)
    agg2 = _agg_feat_split(y2, edge_index)
    y3 = _tc_last(agg2, y2, dinv, b2.reshape(1, HID), W3)
    agg3 = _agg_edge_split(y3, edge_index)
    return _tc_out(agg3, y3, dinv, b3.reshape(1, OUT))


# speedup vs baseline: 43.4827x; 1.5044x over previous
import functools

import jax
import jax.numpy as jnp
from jax import lax
from jax.experimental import pallas as pl
from jax.experimental.pallas import tpu as pltpu
from jax.experimental.pallas import tpu_sc as plsc

N = 100000
E = 1600000
HID = 32
OUT = 16
HALF = 16
ROWS = E // 128
NT = 16
NC = 2
CHUNK_E = 400
RPT = 6256
RPT_LAST = N - 15 * RPT

_mesh = plsc.VectorSubcoreMesh(core_axis_name="c", subcore_axis_name="s")
_sc_params = pltpu.CompilerParams(use_tc_tiling_on_sc=False)


def _zero_acc_2d(rows, acc, t):
    base = t * RPT
    kf = RPT_LAST // CHUNK_E
    for k in range(kf):
        pltpu.sync_copy(rows, acc.at[pl.ds(base + k * CHUNK_E, CHUNK_E)])

    @pl.when(t < 15)
    def _():
        pltpu.sync_copy(rows.at[pl.ds(0, RPT - kf * CHUNK_E)],
                        acc.at[pl.ds(base + kf * CHUNK_E, RPT - kf * CHUNK_E)])

    @pl.when(t == 15)
    def _():
        pltpu.sync_copy(rows.at[pl.ds(0, RPT_LAST - kf * CHUNK_E)],
                        acc.at[pl.ds(base + kf * CHUNK_E,
                                     RPT_LAST - kf * CHUNK_E)])


def _writeout_2d(acc, dst, t):
    base = t * RPT

    @pl.when(t < 15)
    def _():
        pltpu.sync_copy(acc.at[pl.ds(base, RPT)], dst.at[pl.ds(base, RPT)])

    @pl.when(t == 15)
    def _():
        pltpu.sync_copy(acc.at[pl.ds(base, RPT_LAST)],
                        dst.at[pl.ds(base, RPT_LAST)])


def _make_agg(split_edges: bool):
    total_e = E // NC if split_edges else E
    nchunks = total_e // CHUNK_E
    assert nchunks * CHUNK_E == total_e
    base_ch = nchunks // NT
    extra_ch = nchunks % NT

    @functools.partial(
        pl.kernel,
        mesh=_mesh,
        out_type=jax.ShapeDtypeStruct((NC, N, HALF), jnp.float32),
        scratch_types=[
            pltpu.VMEM((2, CHUNK_E), jnp.int32),
            pltpu.VMEM((2, CHUNK_E), jnp.int32),
            pltpu.VMEM((2, CHUNK_E), jnp.int32),
            pltpu.VMEM((CHUNK_E, HALF), jnp.float32),
            pltpu.VMEM((CHUNK_E, HALF), jnp.float32),
            pltpu.VMEM((CHUNK_E, HALF), jnp.float32),
            pltpu.VMEM_SHARED((N, HALF), jnp.float32),
            pltpu.SemaphoreType.DMA,
            pltpu.SemaphoreType.DMA,
            pltpu.SemaphoreType.DMA,
            pltpu.SemaphoreType.DMA,
            pltpu.SemaphoreType.DMA,
            pltpu.SemaphoreType.DMA,
            pltpu.SemaphoreType.DMA,
            pltpu.SemaphoreType.DMA,
            pltpu.SemaphoreType.DMA,
        ],
        compiler_params=_sc_params,
    )
    def agg(table, ei, out, idx0, idx1, idx2, rows0, rows1, rows2, acc,
            xsem0, xsem1, xsem2, gsem0, gsem1, gsem2, ssem0, ssem1, ssem2):
        c = lax.axis_index("c")
        t = lax.axis_index("s")
        tbl = table if split_edges else table.at[c]
        e0 = c * total_e if split_edges else 0

        def zb(i, carry):
            rows0[i, :] = jnp.zeros((HALF,), jnp.float32)
            return carry
        lax.fori_loop(0, CHUNK_E, zb, None)
        _zero_acc_2d(rows0, acc, t)
        plsc.subcore_barrier()

        nmine = base_ch + jnp.where(t < extra_ch, 1, 0)
        SL = ((idx0, rows0, xsem0, gsem0, ssem0),
              (idx1, rows1, xsem1, gsem1, ssem1),
              (idx2, rows2, xsem2, gsem2, ssem2))

        def chunk_off(i):
            return e0 + (t + i * NT) * CHUNK_E

        def load_idx(i, slot):
            pltpu.async_copy(ei.at[:, pl.ds(chunk_off(i), CHUNK_E)],
                             slot[0], slot[2])

        def gather_of(slot):
            return pltpu.make_async_copy(tbl.at[slot[0].at[0]], slot[1],
                                         slot[3])

        def scatter_of(slot):
            return pltpu.make_async_copy(slot[1], acc.at[slot[0].at[1]],
                                         slot[4])

        load_idx(0, SL[0])

        def step(i, cur, prv, nxt):
            @pl.when(i >= 1)
            def _():
                gather_of(prv).wait()
                scatter_of(prv).start(add=True)

            @pl.when(i >= 2)
            def _():
                scatter_of(nxt).wait()

            @pl.when(i + 1 < nmine)
            def _():
                load_idx(i + 1, nxt)

            pltpu.make_async_copy(ei.at[:, pl.ds(0, CHUNK_E)], cur[0],
                                  cur[2]).wait()
            gather_of(cur).start()

        def chunk(i, carry):
            for j in range(3):
                @pl.when(i % 3 == j)
                def _(j=j):
                    step(i, SL[j], SL[(j + 2) % 3], SL[(j + 1) % 3])
            return carry
        lax.fori_loop(0, nmine, chunk, None)

        def fin(cur, prv):
            gather_of(cur).wait()
            scatter_of(cur).start(add=True)

            @pl.when(nmine >= 2)
            def _():
                scatter_of(prv).wait()
            scatter_of(cur).wait()

        for j in range(3):
            @pl.when(nmine % 3 == (j + 1) % 3)
            def _(j=j):
                fin(SL[j], SL[(j + 2) % 3])

        plsc.subcore_barrier()
        _writeout_2d(acc, out.at[c], t)

    return agg


_agg_feat_split = _make_agg(split_edges=False)
_agg_edge_split = _make_agg(split_edges=True)


_DEG_E = E // NC
_DEG_NCH = _DEG_E // CHUNK_E


@functools.partial(
    pl.kernel,
    mesh=_mesh,
    out_type=jax.ShapeDtypeStruct((NC, N, HALF), jnp.float32),
    scratch_types=[
        pltpu.VMEM((CHUNK_E,), jnp.int32),
        pltpu.VMEM((CHUNK_E, HALF), jnp.float32),
        pltpu.VMEM_SHARED((N, HALF), jnp.float32),
    ],
    compiler_params=_sc_params,
)
def _deg_kernel(ei, out, didx0, ones, acc):
    c = lax.axis_index("c")
    t = lax.axis_index("s")
    e0 = c * _DEG_E

    def fill(i, carry):
        ones[i, :] = jnp.zeros((HALF,), jnp.float32)
        return carry
    lax.fori_loop(0, CHUNK_E, fill, None)
    _zero_acc_2d(ones, acc, t)
    plsc.subcore_barrier()

    def refill(i, carry):
        ones[i, :] = jnp.ones((HALF,), jnp.float32)
        return carry
    lax.fori_loop(0, CHUNK_E, refill, None)

    nmine = _DEG_NCH // NT + jnp.where(t < _DEG_NCH % NT, 1, 0)

    def chunk(i, carry):
        eoff = e0 + (t + i * NT) * CHUNK_E
        pltpu.sync_copy(ei.at[1].at[pl.ds(eoff, CHUNK_E)], didx0)
        pltpu.sync_copy(ones, acc.at[didx0], add=True)
        return carry
    lax.fori_loop(0, nmine, chunk, None)

    plsc.subcore_barrier()
    _writeout_2d(acc, out.at[c], t)


_PB = 500
_GRID = (N // 8) // _PB
_F32 = jnp.float32


def _tc1_body(x_ref, degp_ref, w_ref, y_ref, dinv_ref):
    deg = degp_ref[0, 0] + degp_ref[1, 0] + 1.0
    dinv = lax.rsqrt(deg)
    dinv_ref[0] = dinv
    x = x_ref[0]
    y_ref[0, 0] = dinv * jnp.dot(x, w_ref[0], preferred_element_type=_F32)
    y_ref[1, 0] = dinv * jnp.dot(x, w_ref[1], preferred_element_type=_F32)


def _tc_mid_body(agg_ref, yp_ref, dinv_ref, b_ref, w_ref, y_ref):
    dinv = dinv_ref[0]
    aa = jnp.maximum(dinv * (agg_ref[0, 0] + yp_ref[0, 0]) + b_ref[0], 0.0)
    ab = jnp.maximum(dinv * (agg_ref[1, 0] + yp_ref[1, 0]) + b_ref[1], 0.0)
    y_ref[0, 0] = dinv * (jnp.dot(aa, w_ref[0], preferred_element_type=_F32)
                          + jnp.dot(ab, w_ref[1],
                                    preferred_element_type=_F32))
    y_ref[1, 0] = dinv * (jnp.dot(aa, w_ref[2], preferred_element_type=_F32)
                          + jnp.dot(ab, w_ref[3],
                                    preferred_element_type=_F32))


def _tc_last_body(agg_ref, yp_ref, dinv_ref, b_ref, w_ref, y_ref):
    dinv = dinv_ref[0]
    aa = jnp.maximum(dinv * (agg_ref[0, 0] + yp_ref[0, 0]) + b_ref[0], 0.0)
    ab = jnp.maximum(dinv * (agg_ref[1, 0] + yp_ref[1, 0]) + b_ref[1], 0.0)
    y_ref[0] = dinv * (jnp.dot(aa, w_ref[0], preferred_element_type=_F32)
                       + jnp.dot(ab, w_ref[1], preferred_element_type=_F32))


def _tc_out_body(agg_ref, y3_ref, dinv_ref, b_ref, out_ref):
    out_ref[0] = (dinv_ref[0] * (agg_ref[0, 0] + agg_ref[1, 0] + y3_ref[0])
                  + b_ref[...])


def _bsp():
    return pl.BlockSpec((1, _PB, 128), lambda i: (i, 0, 0))


def _bs2p():
    return pl.BlockSpec((NC, 1, _PB, 128), lambda i: (0, i, 0, 0))


def _bsfull(shape):
    return pl.BlockSpec(shape, lambda i: (0,) * len(shape))


_P3 = (_GRID, _PB, 128)
_P4 = (NC, _GRID, _PB, 128)

_tc1 = pl.pallas_call(
    _tc1_body,
    grid=(_GRID,),
    in_specs=[_bsp(), _bs2p(), _bsfull((2, 128, 128))],
    out_specs=[_bs2p(), _bsp()],
    out_shape=[jax.ShapeDtypeStruct(_P4, _F32),
               jax.ShapeDtypeStruct(_P3, _F32)],
)

_tc_mid = pl.pallas_call(
    _tc_mid_body,
    grid=(_GRID,),
    in_specs=[_bs2p(), _bs2p(), _bsp(), _bsfull((2, 1, 128)),
              _bsfull((4, 128, 128))],
    out_specs=_bs2p(),
    out_shape=jax.ShapeDtypeStruct(_P4, _F32),
)

_tc_last = pl.pallas_call(
    _tc_last_body,
    grid=(_GRID,),
    in_specs=[_bs2p(), _bs2p(), _bsp(), _bsfull((2, 1, 128)),
              _bsfull((2, 128, 128))],
    out_specs=_bsp(),
    out_shape=jax.ShapeDtypeStruct(_P3, _F32),
)

_tc_out = pl.pallas_call(
    _tc_out_body,
    grid=(_GRID,),
    in_specs=[_bs2p(), _bsp(), _bsp(), _bsfull((1, 128))],
    out_specs=_bsp(),
    out_shape=jax.ShapeDtypeStruct(_P3, _F32),
)


def kernel(x, edge_index, W1, b1, W2, b2, W3, b3):
    eye8 = jnp.eye(8, dtype=_F32)

    def bd(m):
        return jnp.kron(eye8, m)

    w1p = jnp.pad(W1, ((0, HALF - W1.shape[0]), (0, 0)))
    bd1 = jnp.stack([bd(w1p[:, :HALF]), bd(w1p[:, HALF:])])
    bd2 = jnp.stack([bd(W2[:HALF, :HALF]), bd(W2[HALF:, :HALF]),
                     bd(W2[:HALF, HALF:]), bd(W2[HALF:, HALF:])])
    bd3 = jnp.stack([bd(W3[:HALF, :]), bd(W3[HALF:, :])])
    b1p = jnp.stack([jnp.tile(b1[:HALF], 8), jnp.tile(b1[HALF:], 8)])
    b1p = b1p.reshape(2, 1, 128)
    b2p = jnp.stack([jnp.tile(b2[:HALF], 8), jnp.tile(b2[HALF:], 8)])
    b2p = b2p.reshape(2, 1, 128)
    b3p = jnp.tile(b3, 8).reshape(1, 128)
    x_p = jnp.pad(x, ((0, 0), (0, HALF - x.shape[1]))).reshape(_P3)

    deg16 = _deg_kernel(edge_index)
    degp = deg16.reshape(_P4)
    y1p, dinvp = _tc1(x_p, degp, bd1)
    agg1 = _agg_feat_split(y1p.reshape(NC, N, HALF), edge_index)
    y2p = _tc_mid(agg1.reshape(_P4), y1p, dinvp, b1p, bd2)
    agg2 = _agg_feat_split(y2p.reshape(NC, N, HALF), edge_index)
    y3p = _tc_last(agg2.reshape(_P4), y2p, dinvp, b2p, bd3)
    agg3 = _agg_edge_split(y3p.reshape(N, OUT), edge_index)
    outp = _tc_out(agg3.reshape(_P4), y3p, dinvp, b3p)
    return outp.reshape(N, OUT)


# speedup vs baseline: 54.0724x; 1.2435x over previous
import functools

import jax
import jax.numpy as jnp
from jax import lax
from jax.experimental import pallas as pl
from jax.experimental.pallas import tpu as pltpu
from jax.experimental.pallas import tpu_sc as plsc

N = 100000
E = 1600000
HID = 32
OUT = 16
HALF = 16
ROWS = E // 128
NT = 16
NC = 2
CHUNK_E = 800
RPT = 6256
RPT_LAST = N - 15 * RPT

_mesh = plsc.VectorSubcoreMesh(core_axis_name="c", subcore_axis_name="s")
_sc_params = pltpu.CompilerParams(use_tc_tiling_on_sc=False)


def _zero_acc_2d(rows, acc, t):
    base = t * RPT
    kf = RPT_LAST // CHUNK_E
    for k in range(kf):
        pltpu.sync_copy(rows, acc.at[pl.ds(base + k * CHUNK_E, CHUNK_E)])

    @pl.when(t < 15)
    def _():
        pltpu.sync_copy(rows.at[pl.ds(0, RPT - kf * CHUNK_E)],
                        acc.at[pl.ds(base + kf * CHUNK_E, RPT - kf * CHUNK_E)])

    @pl.when(t == 15)
    def _():
        pltpu.sync_copy(rows.at[pl.ds(0, RPT_LAST - kf * CHUNK_E)],
                        acc.at[pl.ds(base + kf * CHUNK_E,
                                     RPT_LAST - kf * CHUNK_E)])


def _writeout_2d(acc, dst, t):
    base = t * RPT

    @pl.when(t < 15)
    def _():
        pltpu.sync_copy(acc.at[pl.ds(base, RPT)], dst.at[pl.ds(base, RPT)])

    @pl.when(t == 15)
    def _():
        pltpu.sync_copy(acc.at[pl.ds(base, RPT_LAST)],
                        dst.at[pl.ds(base, RPT_LAST)])


def _make_agg(split_edges: bool):
    total_e = E // NC if split_edges else E
    nchunks = total_e // CHUNK_E
    assert nchunks * CHUNK_E == total_e
    base_ch = nchunks // NT
    extra_ch = nchunks % NT

    @functools.partial(
        pl.kernel,
        mesh=_mesh,
        out_type=jax.ShapeDtypeStruct((NC, N, HALF), jnp.float32),
        scratch_types=[
            pltpu.VMEM((2, CHUNK_E), jnp.int32),
            pltpu.VMEM((2, CHUNK_E), jnp.int32),
            pltpu.VMEM((2, CHUNK_E), jnp.int32),
            pltpu.VMEM((CHUNK_E, HALF), jnp.float32),
            pltpu.VMEM((CHUNK_E, HALF), jnp.float32),
            pltpu.VMEM_SHARED((N, HALF), jnp.float32),
            pltpu.SemaphoreType.DMA,
            pltpu.SemaphoreType.DMA,
            pltpu.SemaphoreType.DMA,
            pltpu.SemaphoreType.DMA,
            pltpu.SemaphoreType.DMA,
            pltpu.SemaphoreType.DMA,
            pltpu.SemaphoreType.DMA,
        ],
        compiler_params=_sc_params,
    )
    def agg(table, ei, out, idx0, idx1, idx2, rows0, rows1, acc,
            xsem0, xsem1, xsem2, gsem0, gsem1, ssem0, ssem1):
        c = lax.axis_index("c")
        t = lax.axis_index("s")
        tbl = table if split_edges else table.at[c]
        e0 = c * total_e if split_edges else 0

        def zb(i, carry):
            rows0[i, :] = jnp.zeros((HALF,), jnp.float32)
            return carry
        lax.fori_loop(0, CHUNK_E, zb, None)
        _zero_acc_2d(rows0, acc, t)
        plsc.subcore_barrier()

        nmine = base_ch + jnp.where(t < extra_ch, 1, 0)
        X = (idx0, idx1, idx2)
        XS = (xsem0, xsem1, xsem2)
        R = (rows0, rows1)
        GS = (gsem0, gsem1)
        SS = (ssem0, ssem1)

        def chunk_off(i):
            return e0 + (t + i * NT) * CHUNK_E

        def load_idx(i, k3):
            pltpu.async_copy(ei.at[:, pl.ds(chunk_off(i), CHUNK_E)],
                             X[k3], XS[k3])

        def gather_of(k3, k2):
            return pltpu.make_async_copy(tbl.at[X[k3].at[0]], R[k2], GS[k2])

        def scatter_of(k3, k2):
            return pltpu.make_async_copy(R[k2], acc.at[X[k3].at[1]], SS[k2])

        load_idx(0, 0)

        def step(i, j):
            @pl.when(i >= 1)
            def _():
                gather_of((j + 2) % 3, (j + 1) % 2).wait()
                scatter_of((j + 2) % 3, (j + 1) % 2).start(add=True)

            @pl.when(i >= 2)
            def _():
                scatter_of((j + 1) % 3, j % 2).wait()

            @pl.when(i + 1 < nmine)
            def _():
                load_idx(i + 1, (j + 1) % 3)

            pltpu.make_async_copy(ei.at[:, pl.ds(0, CHUNK_E)], X[j % 3],
                                  XS[j % 3]).wait()
            gather_of(j % 3, j % 2).start()

        def chunk(i, carry):
            for j in range(6):
                @pl.when(i % 6 == j)
                def _(j=j):
                    step(i, j)
            return carry
        lax.fori_loop(0, nmine, chunk, None)

        def fin(j):
            gather_of(j % 3, j % 2).wait()
            scatter_of(j % 3, j % 2).start(add=True)

            @pl.when(nmine >= 2)
            def _():
                scatter_of((j + 2) % 3, (j + 1) % 2).wait()
            scatter_of(j % 3, j % 2).wait()

        for j in range(6):
            @pl.when(nmine % 6 == (j + 1) % 6)
            def _(j=j):
                fin(j)

        plsc.subcore_barrier()
        _writeout_2d(acc, out.at[c], t)

    return agg


_agg_feat_split = _make_agg(split_edges=False)
_agg_edge_split = _make_agg(split_edges=True)


_DEG_E = E // NC
_DEG_NCH = _DEG_E // CHUNK_E


@functools.partial(
    pl.kernel,
    mesh=_mesh,
    out_type=jax.ShapeDtypeStruct((NC, N, HALF), jnp.float32),
    scratch_types=[
        pltpu.VMEM((CHUNK_E,), jnp.int32),
        pltpu.VMEM((CHUNK_E, HALF), jnp.float32),
        pltpu.VMEM_SHARED((N, HALF), jnp.float32),
    ],
    compiler_params=_sc_params,
)
def _deg_kernel(ei, out, didx0, ones, acc):
    c = lax.axis_index("c")
    t = lax.axis_index("s")
    e0 = c * _DEG_E

    def fill(i, carry):
        ones[i, :] = jnp.zeros((HALF,), jnp.float32)
        return carry
    lax.fori_loop(0, CHUNK_E, fill, None)
    _zero_acc_2d(ones, acc, t)
    plsc.subcore_barrier()

    def refill(i, carry):
        ones[i, :] = jnp.ones((HALF,), jnp.float32)
        return carry
    lax.fori_loop(0, CHUNK_E, refill, None)

    nmine = _DEG_NCH // NT + jnp.where(t < _DEG_NCH % NT, 1, 0)

    def chunk(i, carry):
        eoff = e0 + (t + i * NT) * CHUNK_E
        pltpu.sync_copy(ei.at[1].at[pl.ds(eoff, CHUNK_E)], didx0)
        pltpu.sync_copy(ones, acc.at[didx0], add=True)
        return carry
    lax.fori_loop(0, nmine, chunk, None)

    plsc.subcore_barrier()
    _writeout_2d(acc, out.at[c], t)


_PB = 500
_GRID = (N // 8) // _PB
_F32 = jnp.float32


def _tc1_body(x_ref, degp_ref, w_ref, y_ref, dinv_ref):
    deg = degp_ref[0, 0] + degp_ref[1, 0] + 1.0
    dinv = lax.rsqrt(deg)
    dinv_ref[0] = dinv
    x = x_ref[0]
    y_ref[0, 0] = dinv * jnp.dot(x, w_ref[0], preferred_element_type=_F32)
    y_ref[1, 0] = dinv * jnp.dot(x, w_ref[1], preferred_element_type=_F32)


def _tc_mid_body(agg_ref, yp_ref, dinv_ref, b_ref, w_ref, y_ref):
    dinv = dinv_ref[0]
    aa = jnp.maximum(dinv * (agg_ref[0, 0] + yp_ref[0, 0]) + b_ref[0], 0.0)
    ab = jnp.maximum(dinv * (agg_ref[1, 0] + yp_ref[1, 0]) + b_ref[1], 0.0)
    y_ref[0, 0] = dinv * (jnp.dot(aa, w_ref[0], preferred_element_type=_F32)
                          + jnp.dot(ab, w_ref[1],
                                    preferred_element_type=_F32))
    y_ref[1, 0] = dinv * (jnp.dot(aa, w_ref[2], preferred_element_type=_F32)
                          + jnp.dot(ab, w_ref[3],
                                    preferred_element_type=_F32))


def _tc_last_body(agg_ref, yp_ref, dinv_ref, b_ref, w_ref, y_ref):
    dinv = dinv_ref[0]
    aa = jnp.maximum(dinv * (agg_ref[0, 0] + yp_ref[0, 0]) + b_ref[0], 0.0)
    ab = jnp.maximum(dinv * (agg_ref[1, 0] + yp_ref[1, 0]) + b_ref[1], 0.0)
    y_ref[0] = dinv * (jnp.dot(aa, w_ref[0], preferred_element_type=_F32)
                       + jnp.dot(ab, w_ref[1], preferred_element_type=_F32))


def _tc_out_body(agg_ref, y3_ref, dinv_ref, b_ref, out_ref):
    out_ref[0] = (dinv_ref[0] * (agg_ref[0, 0] + agg_ref[1, 0] + y3_ref[0])
                  + b_ref[...])


def _bsp():
    return pl.BlockSpec((1, _PB, 128), lambda i: (i, 0, 0))


def _bs2p():
    return pl.BlockSpec((NC, 1, _PB, 128), lambda i: (0, i, 0, 0))


def _bsfull(shape):
    return pl.BlockSpec(shape, lambda i: (0,) * len(shape))


_P3 = (_GRID, _PB, 128)
_P4 = (NC, _GRID, _PB, 128)

_tc1 = pl.pallas_call(
    _tc1_body,
    grid=(_GRID,),
    in_specs=[_bsp(), _bs2p(), _bsfull((2, 128, 128))],
    out_specs=[_bs2p(), _bsp()],
    out_shape=[jax.ShapeDtypeStruct(_P4, _F32),
               jax.ShapeDtypeStruct(_P3, _F32)],
)

_tc_mid = pl.pallas_call(
    _tc_mid_body,
    grid=(_GRID,),
    in_specs=[_bs2p(), _bs2p(), _bsp(), _bsfull((2, 1, 128)),
              _bsfull((4, 128, 128))],
    out_specs=_bs2p(),
    out_shape=jax.ShapeDtypeStruct(_P4, _F32),
)

_tc_last = pl.pallas_call(
    _tc_last_body,
    grid=(_GRID,),
    in_specs=[_bs2p(), _bs2p(), _bsp(), _bsfull((2, 1, 128)),
              _bsfull((2, 128, 128))],
    out_specs=_bsp(),
    out_shape=jax.ShapeDtypeStruct(_P3, _F32),
)

_tc_out = pl.pallas_call(
    _tc_out_body,
    grid=(_GRID,),
    in_specs=[_bs2p(), _bsp(), _bsp(), _bsfull((1, 128))],
    out_specs=_bsp(),
    out_shape=jax.ShapeDtypeStruct(_P3, _F32),
)


def kernel(x, edge_index, W1, b1, W2, b2, W3, b3):
    eye8 = jnp.eye(8, dtype=_F32)

    def bd(m):
        return jnp.kron(eye8, m)

    w1p = jnp.pad(W1, ((0, HALF - W1.shape[0]), (0, 0)))
    bd1 = jnp.stack([bd(w1p[:, :HALF]), bd(w1p[:, HALF:])])
    bd2 = jnp.stack([bd(W2[:HALF, :HALF]), bd(W2[HALF:, :HALF]),
                     bd(W2[:HALF, HALF:]), bd(W2[HALF:, HALF:])])
    bd3 = jnp.stack([bd(W3[:HALF, :]), bd(W3[HALF:, :])])
    b1p = jnp.stack([jnp.tile(b1[:HALF], 8), jnp.tile(b1[HALF:], 8)])
    b1p = b1p.reshape(2, 1, 128)
    b2p = jnp.stack([jnp.tile(b2[:HALF], 8), jnp.tile(b2[HALF:], 8)])
    b2p = b2p.reshape(2, 1, 128)
    b3p = jnp.tile(b3, 8).reshape(1, 128)
    x_p = jnp.pad(x, ((0, 0), (0, HALF - x.shape[1]))).reshape(_P3)

    deg16 = _deg_kernel(edge_index)
    degp = deg16.reshape(_P4)
    y1p, dinvp = _tc1(x_p, degp, bd1)
    agg1 = _agg_feat_split(y1p.reshape(NC, N, HALF), edge_index)
    y2p = _tc_mid(agg1.reshape(_P4), y1p, dinvp, b1p, bd2)
    agg2 = _agg_feat_split(y2p.reshape(NC, N, HALF), edge_index)
    y3p = _tc_last(agg2.reshape(_P4), y2p, dinvp, b2p, bd3)
    agg3 = _agg_edge_split(y3p.reshape(N, OUT), edge_index)
    outp = _tc_out(agg3.reshape(_P4), y3p, dinvp, b3p)
    return outp.reshape(N, OUT)
